# baseline (speedup 1.0000x reference)
import functools
import math

import jax
import jax.numpy as jnp
from jax import lax
from jax.experimental import pallas as pl
from jax.experimental.pallas import tpu as pltpu
from jax.experimental.pallas import tpu_sc as plsc

VOCAB = 1000000
D = 32
B = 4096
L = 50

NC = 2
NS = 16
NW = NC * NS
LANES = 16

N_TOTAL = B * L
B_PER_W = N_TOTAL // NW
CHUNK = 128
N_CHUNKS = B_PER_W // CHUNK
NBUF = 5
N_GROUPS = N_CHUNKS // NBUF

SCALE = math.sqrt(D)


@functools.partial(
    pl.kernel,
    out_type=jax.ShapeDtypeStruct((N_TOTAL, D), jnp.float32),
    mesh=plsc.VectorSubcoreMesh(core_axis_name="c", subcore_axis_name="s"),
    scratch_types=[
        pltpu.VMEM((B_PER_W,), jnp.int32),
        *[pltpu.VMEM((CHUNK, D), jnp.float32) for _ in range(NBUF)],
        *[pltpu.SemaphoreType.DMA for _ in range(NBUF)],
    ],
    compiler_params=pltpu.CompilerParams(use_tc_tiling_on_sc=False),
)
def _emb_lookup(x_hbm, table_hbm, out_hbm, idx_v, *bufs_and_sems):
    rows = bufs_and_sems[:NBUF]
    gsem = bufs_and_sems[NBUF:]
    wid = lax.axis_index("s") * NC + lax.axis_index("c")
    base = wid * B_PER_W
    pltpu.sync_copy(x_hbm.at[pl.ds(base, B_PER_W)], idx_v)

    def chunk_idx(c):
        return idx_v.at[pl.ds(c * CHUNK, CHUNK)]

    for b in range(NBUF):
        pltpu.async_copy(table_hbm.at[chunk_idx(b)], rows[b], gsem[b])

    @pl.loop(0, N_GROUPS)
    def _group(g):
        for b in range(NBUF):
            c = g * NBUF + b
            pltpu.make_async_copy(
                table_hbm.at[chunk_idx(c)], rows[b], gsem[b]).wait()

            @pl.loop(0, CHUNK, unroll=8)
            def _row(r):
                for h in range(D // LANES):
                    sl = pl.ds(h * LANES, LANES)
                    rows[b][r, sl] = rows[b][r, sl] * SCALE

            pltpu.sync_copy(
                rows[b], out_hbm.at[pl.ds(base + c * CHUNK, CHUNK)])

            @pl.when(g + 1 < N_GROUPS)
            def _prefetch():
                pltpu.async_copy(
                    table_hbm.at[chunk_idx(c + NBUF)], rows[b], gsem[b])


def kernel(x, table):
    out = _emb_lookup(x.reshape(N_TOTAL), table)
    return out.reshape(B, L, D)


# speedup vs baseline: 1.1538x; 1.1538x over previous
import functools
import math

import jax
import jax.numpy as jnp
from jax import lax
from jax.experimental import pallas as pl
from jax.experimental.pallas import tpu as pltpu
from jax.experimental.pallas import tpu_sc as plsc

VOCAB = 1000000
D = 32
B = 4096
L = 50

NC = 2
NS = 16
NW = NC * NS
LANES = 16

N_TOTAL = B * L
B_PER_W = N_TOTAL // NW
CHUNK = 128
FLAT = CHUNK * D
N_CHUNKS = B_PER_W // CHUNK
NBUF = 5
N_GROUPS = N_CHUNKS // NBUF

SCALE = math.sqrt(D)


@functools.partial(
    pl.kernel,
    out_type=jax.ShapeDtypeStruct((N_TOTAL * D,), jnp.float32),
    mesh=plsc.VectorSubcoreMesh(core_axis_name="c", subcore_axis_name="s"),
    scratch_types=[
        pltpu.VMEM((B_PER_W,), jnp.int32),
        *[pltpu.VMEM((CHUNK, D), jnp.float32) for _ in range(NBUF)],
        *[pltpu.VMEM((FLAT,), jnp.float32) for _ in range(NBUF)],
        *[pltpu.SemaphoreType.DMA for _ in range(2 * NBUF)],
    ],
    compiler_params=pltpu.CompilerParams(use_tc_tiling_on_sc=False),
)
def _emb_lookup(x_hbm, table_hbm, out_hbm, idx_v, *scratch):
    rows = scratch[:NBUF]
    stage = scratch[NBUF:2 * NBUF]
    gsem = scratch[2 * NBUF:3 * NBUF]
    ssem = scratch[3 * NBUF:]
    wid = lax.axis_index("s") * NC + lax.axis_index("c")
    base = wid * B_PER_W
    pltpu.sync_copy(x_hbm.at[pl.ds(base, B_PER_W)], idx_v)

    def chunk_idx(c):
        return idx_v.at[pl.ds(c * CHUNK, CHUNK)]

    def out_at(c):
        return out_hbm.at[pl.ds((base + c * CHUNK) * D, FLAT)]

    for b in range(NBUF):
        pltpu.async_copy(table_hbm.at[chunk_idx(b)], rows[b], gsem[b])

    @pl.loop(0, N_GROUPS)
    def _group(g):
        for b in range(NBUF):
            c = g * NBUF + b
            pltpu.make_async_copy(
                table_hbm.at[chunk_idx(c)], rows[b], gsem[b]).wait()

            @pl.when(g > 0)
            def _stage_free():
                pltpu.make_async_copy(stage[b], out_at(c), ssem[b]).wait()

            @pl.loop(0, CHUNK, unroll=8)
            def _row(r):
                for h in range(D // LANES):
                    stage[b][pl.ds(r * D + h * LANES, LANES)] = (
                        rows[b][r, pl.ds(h * LANES, LANES)] * SCALE)

            @pl.when(g + 1 < N_GROUPS)
            def _prefetch():
                pltpu.async_copy(
                    table_hbm.at[chunk_idx(c + NBUF)], rows[b], gsem[b])

            pltpu.async_copy(stage[b], out_at(c), ssem[b])

    for b in range(NBUF):
        pltpu.make_async_copy(
            stage[b], out_hbm.at[pl.ds(base * D, FLAT)], ssem[b]).wait()


def kernel(x, table):
    out = _emb_lookup(x.reshape(N_TOTAL), table)
    return out.reshape(B, L, D)


# speedup vs baseline: 1.1836x; 1.0259x over previous
import functools
import math

import jax
import jax.numpy as jnp
from jax import lax
from jax.experimental import pallas as pl
from jax.experimental.pallas import tpu as pltpu
from jax.experimental.pallas import tpu_sc as plsc

VOCAB = 1000000
D = 32
B = 4096
L = 50

NC = 2
NS = 16
NW = NC * NS
LANES = 16

N_TOTAL = B * L
B_PER_W = N_TOTAL // NW
CHUNK = 128
FLAT = CHUNK * D
N_CHUNKS = B_PER_W // CHUNK
NBUF = 5
N_GROUPS = N_CHUNKS // NBUF

SCALE = math.sqrt(D)


@functools.partial(
    pl.kernel,
    out_type=jax.ShapeDtypeStruct((N_TOTAL * D,), jnp.float32),
    mesh=plsc.VectorSubcoreMesh(core_axis_name="c", subcore_axis_name="s"),
    scratch_types=[
        pltpu.VMEM((B_PER_W,), jnp.int32),
        *[pltpu.VMEM((CHUNK, D), jnp.float32) for _ in range(NBUF)],
        *[pltpu.VMEM((FLAT,), jnp.float32) for _ in range(NBUF)],
        *[pltpu.SemaphoreType.DMA for _ in range(2 * NBUF)],
    ],
    compiler_params=pltpu.CompilerParams(use_tc_tiling_on_sc=False),
)
def _emb_lookup(x_hbm, table_hbm, out_hbm, idx_v, *scratch):
    rows = scratch[:NBUF]
    stage = scratch[NBUF:2 * NBUF]
    gsem = scratch[2 * NBUF:3 * NBUF]
    ssem = scratch[3 * NBUF:]
    wid = lax.axis_index("s") * NC + lax.axis_index("c")
    base = wid * B_PER_W
    pltpu.sync_copy(x_hbm.at[pl.ds(base, B_PER_W)], idx_v)

    def chunk_idx(c):
        return idx_v.at[pl.ds(c * CHUNK, CHUNK)]

    def out_at(c):
        return out_hbm.at[pl.ds((base + c * CHUNK) * D, FLAT)]

    for b in range(NBUF):
        pltpu.async_copy(table_hbm.at[chunk_idx(b)], rows[b], gsem[b])

    @pl.loop(0, N_GROUPS)
    def _group(g):
        for b in range(NBUF):
            c = g * NBUF + b
            pltpu.make_async_copy(
                table_hbm.at[chunk_idx(c)], rows[b], gsem[b]).wait()

            @pl.when(g > 0)
            def _stage_free():
                pltpu.make_async_copy(stage[b], out_at(c), ssem[b]).wait()

            @pl.loop(0, CHUNK, unroll=8)
            def _row(r):
                for h in range(D // LANES):
                    stage[b][pl.ds(r * D + h * LANES, LANES)] = (
                        rows[b][r, pl.ds(h * LANES, LANES)] * SCALE)

            @pl.when(g + 1 < N_GROUPS)
            def _prefetch():
                pltpu.async_copy(
                    table_hbm.at[chunk_idx(c + NBUF)], rows[b], gsem[b])

            pltpu.async_copy(stage[b], out_at(c), ssem[b])

    for b in range(NBUF):
        pltpu.make_async_copy(
            stage[b], out_hbm.at[pl.ds(base * D, FLAT)], ssem[b]).wait()


def kernel(x, table):
    out = _emb_lookup(x.T.reshape(N_TOTAL), table)
    return out.reshape(L, B, D).transpose(1, 0, 2)
